# baseline (speedup 1.0000x reference)
import jax
import jax.numpy as jnp
from jax import lax
from jax.experimental import pallas as pl
from jax.experimental.pallas import tpu as pltpu
from jax.experimental.pallas import tpu_sc as plsc

VOCAB = 30522
HID = 1024
MAXPOS = 4096
B = 4
S = 4096
N = B * S
L = 16
NC = 2
NS = 16
NW = NC * NS
TPW = N // NW
C = 16
NCHUNK = TPW // C
HJ = HID // L

_DNUMS = lax.GatherDimensionNumbers(
    offset_dims=(), collapsed_slice_dims=(0,), start_index_map=(0,))


def _dyn_gather(v, idx):
    return lax.gather(v, idx[:, None], _DNUMS, (1,),
                      mode=lax.GatherScatterMode.PROMISE_IN_BOUNDS)


def _lane_sum_bcast(v):
    idx = lax.iota(jnp.int32, L)
    for d in (1, 2, 4, 8):
        v = v + _dyn_gather(v, idx ^ d)
    return v


def _rsqrt_newton(v):
    i = lax.bitcast_convert_type(v, jnp.int32)
    y = lax.bitcast_convert_type(jnp.int32(0x5F3759DF) - (i >> 1), jnp.float32)
    vh = v * jnp.float32(0.5)
    for _ in range(3):
        y = y * (jnp.float32(1.5) - vh * y * y)
    return y


def _sc_body(ids_hbm, tts_hbm, word_hbm, pos_hbm, ttemb_hbm, gamma_hbm,
             beta_hbm, out_hbm, idx_v, ttid_v, w_v, p_v, t_v, g_v, b_v,
             sem_w, sem_t):
    wid = lax.axis_index("s") * NC + lax.axis_index("c")
    base = wid * TPW
    pos_base = base % S

    pltpu.sync_copy(gamma_hbm, g_v)
    pltpu.sync_copy(beta_hbm, b_v)

    def chunk_body(c, _):
        tbase = base + c * C
        pltpu.sync_copy(ids_hbm.at[pl.ds(tbase, C)], idx_v)
        pltpu.sync_copy(tts_hbm.at[pl.ds(tbase, C)], ttid_v)
        cp_w = pltpu.async_copy(word_hbm.at[idx_v], w_v, sem_w)
        cp_t = pltpu.async_copy(ttemb_hbm.at[ttid_v], t_v, sem_t)
        pltpu.sync_copy(pos_hbm.at[pl.ds(pos_base + c * C, C)], p_v)
        cp_w.wait()
        cp_t.wait()

        def tok_body(t, _):
            def sum_body(j, carry):
                s_v, q_v = carry
                sl = pl.ds(j * L, L)
                e = w_v[t, sl] + p_v[t, sl] + t_v[t, sl]
                w_v[t, sl] = e
                return s_v + e, q_v + e * e

            zeros = jnp.zeros((L,), jnp.float32)
            s_v, q_v = lax.fori_loop(0, HJ, sum_body, (zeros, zeros))
            mean_v = _lane_sum_bcast(s_v) * jnp.float32(1.0 / HID)
            msq_v = _lane_sum_bcast(q_v) * jnp.float32(1.0 / HID)
            var_v = msq_v - mean_v * mean_v + jnp.float32(1e-12)
            rstd_v = _rsqrt_newton(var_v)
            mrs_v = mean_v * rstd_v

            def norm_body(j, _):
                sl = pl.ds(j * L, L)
                e = w_v[t, sl]
                o = e * rstd_v - mrs_v
                w_v[t, sl] = o * g_v[sl] + b_v[sl]
                return 0

            lax.fori_loop(0, HJ, norm_body, 0)
            return 0

        lax.fori_loop(0, C, tok_body, 0)
        pltpu.sync_copy(w_v, out_hbm.at[pl.ds(tbase, C)])
        return 0

    lax.fori_loop(0, NCHUNK, chunk_body, 0)


@jax.jit
def kernel(input_ids, token_type_ids, word_embeddings, position_embeddings,
           token_type_embeddings, ln_gamma, ln_beta):
    ids = input_ids.reshape(N).astype(jnp.int32)
    tts = token_type_ids.reshape(N).astype(jnp.int32)

    emb = pl.kernel(
        _sc_body,
        out_type=jax.ShapeDtypeStruct((N, HID), jnp.float32),
        mesh=plsc.VectorSubcoreMesh(core_axis_name="c", subcore_axis_name="s"),
        scratch_types=[
            pltpu.VMEM((C,), jnp.int32),
            pltpu.VMEM((C,), jnp.int32),
            pltpu.VMEM((C, HID), jnp.float32),
            pltpu.VMEM((C, HID), jnp.float32),
            pltpu.VMEM((C, HID), jnp.float32),
            pltpu.VMEM((HID,), jnp.float32),
            pltpu.VMEM((HID,), jnp.float32),
            pltpu.SemaphoreType.DMA,
            pltpu.SemaphoreType.DMA,
        ],
    )(ids, tts, word_embeddings, position_embeddings, token_type_embeddings,
      ln_gamma, ln_beta)
    return emb.reshape(B, S, HID)


# speedup vs baseline: 2.4001x; 2.4001x over previous
import jax
import jax.numpy as jnp
from jax import lax
from jax.experimental import pallas as pl
from jax.experimental.pallas import tpu as pltpu
from jax.experimental.pallas import tpu_sc as plsc

VOCAB = 30522
HID = 1024
MAXPOS = 4096
B = 4
S = 4096
N = B * S
L = 16
NC = 2
NS = 16
NW = NC * NS
TPW = N // NW
C = 16
NCHUNK = TPW // C
HJ = HID // L
HALF = C // 2

_DNUMS = lax.GatherDimensionNumbers(
    offset_dims=(), collapsed_slice_dims=(0,), start_index_map=(0,))


def _dyn_gather(v, idx):
    return lax.gather(v, idx[:, None], _DNUMS, (1,),
                      mode=lax.GatherScatterMode.PROMISE_IN_BOUNDS)


def _lane_sum_bcast(v):
    idx = lax.iota(jnp.int32, L)
    for d in (1, 2, 4, 8):
        v = v + _dyn_gather(v, idx ^ d)
    return v


def _lane_bcast(v, t):
    return _dyn_gather(v, jnp.full((L,), t, jnp.int32))


def _rsqrt_newton(v):
    i = lax.bitcast_convert_type(v, jnp.int32)
    y = lax.bitcast_convert_type(jnp.int32(0x5F3759DF) - (i >> 1), jnp.float32)
    vh = v * jnp.float32(0.5)
    for _ in range(3):
        y = y * (jnp.float32(1.5) - vh * y * y)
    return y


def _sc_body(ids_hbm, tts_hbm, word_hbm, pos_hbm, ttemb_hbm, gamma_hbm,
             beta_hbm, out_hbm, ids_v, tts_v, w_v, p_v, o_v, g_v, b_v,
             tte_v, sem_w, sem_p):
    wid = lax.axis_index("s") * NC + lax.axis_index("c")
    base = wid * TPW
    pos_base = base % S

    pltpu.sync_copy(gamma_hbm, g_v)
    pltpu.sync_copy(beta_hbm, b_v)
    pltpu.sync_copy(ttemb_hbm, tte_v)
    pltpu.sync_copy(ids_hbm.at[pl.ds(base, TPW)], ids_v)
    pltpu.sync_copy(tts_hbm.at[pl.ds(base, TPW)], tts_v)

    zero = jnp.zeros((L,), jnp.float32)

    def chunk_body(c, _):
        tbase = base + c * C
        idx16 = ids_v[pl.ds(c * C, C)]
        cp_w = pltpu.async_copy(word_hbm.at[idx16], w_v, sem_w)
        cp_p = pltpu.async_copy(
            pos_hbm.at[pl.ds(pos_base + c * C, C)], p_v, sem_p)
        cp_w.wait()
        cp_p.wait()

        ttf16 = lax.convert_element_type(tts_v[pl.ds(c * C, C)], jnp.float32)

        for h0 in (0, HALF):
            ttf = [_lane_bcast(ttf16, h0 + i) for i in range(HALF)]

            def p1_body(j, acc, ttf=ttf, h0=h0):
                sl = pl.ds(j * L, L)
                t0 = tte_v[0, sl]
                td = tte_v[1, sl] - t0
                out = []
                for i in range(HALF):
                    t = h0 + i
                    e = w_v[t, sl] + p_v[t, sl] + (t0 + ttf[i] * td)
                    w_v[t, sl] = e
                    out.append(acc[2 * i] + e)
                    out.append(acc[2 * i + 1] + e * e)
                return tuple(out)

            acc = lax.fori_loop(0, HJ, p1_body,
                                tuple(zero for _ in range(2 * HALF)))

            rs = []
            ms = []
            for i in range(HALF):
                mean_v = _lane_sum_bcast(acc[2 * i]) * jnp.float32(1.0 / HID)
                msq_v = _lane_sum_bcast(acc[2 * i + 1]) * jnp.float32(1.0 / HID)
                var_v = msq_v - mean_v * mean_v + jnp.float32(1e-12)
                r = _rsqrt_newton(var_v)
                rs.append(r)
                ms.append(mean_v * r)

            def p3_body(j, _, rs=rs, ms=ms, h0=h0):
                sl = pl.ds(j * L, L)
                g = g_v[sl]
                bb = b_v[sl]
                for i in range(HALF):
                    t = h0 + i
                    e = w_v[t, sl]
                    o_v[t, sl] = (e * rs[i] - ms[i]) * g + bb
                return 0

            lax.fori_loop(0, HJ, p3_body, 0)

        pltpu.sync_copy(o_v, out_hbm.at[pl.ds(tbase, C)])
        return 0

    lax.fori_loop(0, NCHUNK, chunk_body, 0)


@jax.jit
def kernel(input_ids, token_type_ids, word_embeddings, position_embeddings,
           token_type_embeddings, ln_gamma, ln_beta):
    ids = input_ids.reshape(N).astype(jnp.int32)
    tts = token_type_ids.reshape(N).astype(jnp.int32)

    emb = pl.kernel(
        _sc_body,
        out_type=jax.ShapeDtypeStruct((N, HID), jnp.float32),
        mesh=plsc.VectorSubcoreMesh(core_axis_name="c", subcore_axis_name="s"),
        scratch_types=[
            pltpu.VMEM((TPW,), jnp.int32),
            pltpu.VMEM((TPW,), jnp.int32),
            pltpu.VMEM((C, HID), jnp.float32),
            pltpu.VMEM((C, HID), jnp.float32),
            pltpu.VMEM((C, HID), jnp.float32),
            pltpu.VMEM((HID,), jnp.float32),
            pltpu.VMEM((HID,), jnp.float32),
            pltpu.VMEM((2, HID), jnp.float32),
            pltpu.SemaphoreType.DMA,
            pltpu.SemaphoreType.DMA,
        ],
    )(ids, tts, word_embeddings, position_embeddings, token_type_embeddings,
      ln_gamma, ln_beta)
    return emb.reshape(B, S, HID)


# speedup vs baseline: 3.8369x; 1.5986x over previous
import jax
import jax.numpy as jnp
from jax import lax
from jax.experimental import pallas as pl
from jax.experimental.pallas import tpu as pltpu
from jax.experimental.pallas import tpu_sc as plsc

VOCAB = 30522
HID = 1024
MAXPOS = 4096
B = 4
S = 4096
N = B * S
L = 16
NC = 2
NS = 16
NW = NC * NS
TPW = N // NW
C = 16
NCHUNK = TPW // C
HJ = HID // L
HALF = C // 2

_DNUMS = lax.GatherDimensionNumbers(
    offset_dims=(), collapsed_slice_dims=(0,), start_index_map=(0,))


def _dyn_gather(v, idx):
    return lax.gather(v, idx[:, None], _DNUMS, (1,),
                      mode=lax.GatherScatterMode.PROMISE_IN_BOUNDS)


def _lane_sum_bcast(v):
    idx = lax.iota(jnp.int32, L)
    for d in (1, 2, 4, 8):
        v = v + _dyn_gather(v, idx ^ d)
    return v


def _lane_bcast(v, t):
    return _dyn_gather(v, jnp.full((L,), t, jnp.int32))


def _rsqrt_newton(v):
    i = lax.bitcast_convert_type(v, jnp.int32)
    y = lax.bitcast_convert_type(jnp.int32(0x5F3759DF) - (i >> 1), jnp.float32)
    vh = v * jnp.float32(0.5)
    for _ in range(3):
        y = y * (jnp.float32(1.5) - vh * y * y)
    return y


def _sc_body(ids_hbm, tts_hbm, word_hbm, pos_hbm, ttemb_hbm, gamma_hbm,
             beta_hbm, out_hbm, ids_v, tts_v, w_v, p_v, o_v, g_v, b_v,
             tte_v, sem_w0, sem_w1, sem_p0, sem_p1, sem_o0, sem_o1):
    wid = lax.axis_index("s") * NC + lax.axis_index("c")
    base = wid * TPW
    pos_base = base % S

    sem_w = (sem_w0, sem_w1)
    sem_p = (sem_p0, sem_p1)
    sem_o = (sem_o0, sem_o1)

    pltpu.sync_copy(gamma_hbm, g_v)
    pltpu.sync_copy(beta_hbm, b_v)
    pltpu.sync_copy(ttemb_hbm, tte_v)
    pltpu.sync_copy(ids_hbm.at[pl.ds(base, TPW)], ids_v)
    pltpu.sync_copy(tts_hbm.at[pl.ds(base, TPW)], tts_v)

    zero = jnp.zeros((L,), jnp.float32)

    def issue_in(n, b):
        idx16 = ids_v[pl.ds(n * C, C)]
        pltpu.async_copy(word_hbm.at[idx16], w_v.at[b], sem_w[b])
        pltpu.async_copy(pos_hbm.at[pl.ds(pos_base + n * C, C)], p_v.at[b],
                         sem_p[b])

    def wait_in(b):
        pltpu.make_async_copy(
            pos_hbm.at[pl.ds(pos_base, C)], w_v.at[b], sem_w[b]).wait()
        pltpu.make_async_copy(
            pos_hbm.at[pl.ds(pos_base, C)], p_v.at[b], sem_p[b]).wait()

    def wait_out(b):
        pltpu.make_async_copy(
            o_v.at[b], out_hbm.at[pl.ds(base, C)], sem_o[b]).wait()

    issue_in(0, 0)
    issue_in(1, 1)

    def pair_body(m, _):
        for b in (0, 1):
            n = 2 * m + b
            tbase = base + n * C
            wait_in(b)

            @pl.when(m >= 1)
            def _():
                wait_out(b)

            ttf16 = lax.convert_element_type(
                tts_v[pl.ds(n * C, C)], jnp.float32)

            accs = []
            for h0 in (0, HALF):
                ttf = [_lane_bcast(ttf16, h0 + i) for i in range(HALF)]

                def p1_body(j, acc, ttf=ttf, h0=h0, b=b):
                    sl = pl.ds(j * L, L)
                    t0 = tte_v[0, sl]
                    td = tte_v[1, sl] - t0
                    out = []
                    for i in range(HALF):
                        t = h0 + i
                        e = w_v[b, t, sl] + p_v[b, t, sl] + (t0 + ttf[i] * td)
                        o_v[b, t, sl] = e
                        out.append(acc[2 * i] + e)
                        out.append(acc[2 * i + 1] + e * e)
                    return tuple(out)

                accs.append(lax.fori_loop(0, HJ, p1_body,
                                          tuple(zero for _ in range(2 * HALF))))

            @pl.when(n + 2 < NCHUNK)
            def _():
                issue_in(n + 2, b)

            for h0, acc in zip((0, HALF), accs):
                rs = []
                ms = []
                for i in range(HALF):
                    mean_v = (_lane_sum_bcast(acc[2 * i])
                              * jnp.float32(1.0 / HID))
                    msq_v = (_lane_sum_bcast(acc[2 * i + 1])
                             * jnp.float32(1.0 / HID))
                    var_v = msq_v - mean_v * mean_v + jnp.float32(1e-12)
                    r = _rsqrt_newton(var_v)
                    rs.append(r)
                    ms.append(mean_v * r)

                def p3_body(j, _, rs=rs, ms=ms, h0=h0, b=b):
                    sl = pl.ds(j * L, L)
                    g = g_v[sl]
                    bb = b_v[sl]
                    for i in range(HALF):
                        t = h0 + i
                        e = o_v[b, t, sl]
                        o_v[b, t, sl] = (e * rs[i] - ms[i]) * g + bb
                    return 0

                lax.fori_loop(0, HJ, p3_body, 0)

            pltpu.async_copy(o_v.at[b], out_hbm.at[pl.ds(tbase, C)], sem_o[b])
        return 0

    lax.fori_loop(0, NCHUNK // 2, pair_body, 0)
    wait_out(0)
    wait_out(1)


@jax.jit
def kernel(input_ids, token_type_ids, word_embeddings, position_embeddings,
           token_type_embeddings, ln_gamma, ln_beta):
    ids = input_ids.reshape(N).astype(jnp.int32)
    tts = token_type_ids.reshape(N).astype(jnp.int32)

    emb = pl.kernel(
        _sc_body,
        out_type=jax.ShapeDtypeStruct((N, HID), jnp.float32),
        mesh=plsc.VectorSubcoreMesh(core_axis_name="c", subcore_axis_name="s"),
        scratch_types=[
            pltpu.VMEM((TPW,), jnp.int32),
            pltpu.VMEM((TPW,), jnp.int32),
            pltpu.VMEM((2, C, HID), jnp.float32),
            pltpu.VMEM((2, C, HID), jnp.float32),
            pltpu.VMEM((2, C, HID), jnp.float32),
            pltpu.VMEM((HID,), jnp.float32),
            pltpu.VMEM((HID,), jnp.float32),
            pltpu.VMEM((2, HID), jnp.float32),
            pltpu.SemaphoreType.DMA,
            pltpu.SemaphoreType.DMA,
            pltpu.SemaphoreType.DMA,
            pltpu.SemaphoreType.DMA,
            pltpu.SemaphoreType.DMA,
            pltpu.SemaphoreType.DMA,
        ],
    )(ids, tts, word_embeddings, position_embeddings, token_type_embeddings,
      ln_gamma, ln_beta)
    return emb.reshape(B, S, HID)


# speedup vs baseline: 4.7280x; 1.2323x over previous
import jax
import jax.numpy as jnp
from jax import lax
from jax.experimental import pallas as pl
from jax.experimental.pallas import tpu as pltpu
from jax.experimental.pallas import tpu_sc as plsc

VOCAB = 30522
HID = 1024
MAXPOS = 4096
B = 4
S = 4096
N = B * S
L = 16
NC = 2
NS = 16
NW = NC * NS
TPW = N // NW
C = 16
NCHUNK = TPW // C
HJ = HID // L
HALF = C // 2

_DNUMS = lax.GatherDimensionNumbers(
    offset_dims=(), collapsed_slice_dims=(0,), start_index_map=(0,))


def _dyn_gather(v, idx):
    return lax.gather(v, idx[:, None], _DNUMS, (1,),
                      mode=lax.GatherScatterMode.PROMISE_IN_BOUNDS)


def _lane_sum_bcast(v):
    idx = lax.iota(jnp.int32, L)
    for d in (1, 2, 4, 8):
        v = v + _dyn_gather(v, idx ^ d)
    return v


def _lane_bcast(v, t):
    return _dyn_gather(v, jnp.full((L,), t, jnp.int32))


def _rsqrt_newton(v):
    i = lax.bitcast_convert_type(v, jnp.int32)
    y = lax.bitcast_convert_type(jnp.int32(0x5F3759DF) - (i >> 1), jnp.float32)
    vh = v * jnp.float32(0.5)
    for _ in range(3):
        y = y * (jnp.float32(1.5) - vh * y * y)
    return y


def _sc_body(ids_hbm, tts_hbm, word_hbm, pos_hbm, ttemb_hbm, gamma_hbm,
             beta_hbm, out_hbm, ids_v, tts_v, w_v, p_v, o_v, g_v, b_v,
             tte_v, sem_w0, sem_w1, sem_p0, sem_p1, sem_o0, sem_o1):
    wid = lax.axis_index("s") * NC + lax.axis_index("c")
    base = wid * TPW
    pos_base = base % S

    sem_w = (sem_w0, sem_w1)
    sem_p = (sem_p0, sem_p1)
    sem_o = (sem_o0, sem_o1)

    pltpu.sync_copy(gamma_hbm, g_v)
    pltpu.sync_copy(beta_hbm, b_v)
    pltpu.sync_copy(ttemb_hbm, tte_v)
    pltpu.sync_copy(ids_hbm.at[pl.ds(base, TPW)], ids_v)
    pltpu.sync_copy(tts_hbm.at[pl.ds(base, TPW)], tts_v)

    zero = jnp.zeros((L,), jnp.float32)

    def issue_in(n, b):
        idx16 = ids_v[pl.ds(n * C, C)]
        pltpu.async_copy(word_hbm.at[idx16], w_v.at[b], sem_w[b])
        pltpu.async_copy(pos_hbm.at[pl.ds(pos_base + n * C, C)], p_v.at[b],
                         sem_p[b])

    def wait_in(b):
        pltpu.make_async_copy(
            pos_hbm.at[pl.ds(pos_base, C)], w_v.at[b], sem_w[b]).wait()
        pltpu.make_async_copy(
            pos_hbm.at[pl.ds(pos_base, C)], p_v.at[b], sem_p[b]).wait()

    def wait_out(b):
        pltpu.make_async_copy(
            o_v.at[b], out_hbm.at[pl.ds(base, C)], sem_o[b]).wait()

    issue_in(0, 0)
    issue_in(1, 1)

    def pair_body(m, _):
        for b in (0, 1):
            n = 2 * m + b
            tbase = base + n * C
            wait_in(b)

            @pl.when(m >= 1)
            def _():
                wait_out(b)

            ttf16 = lax.convert_element_type(tts_v[pl.ds(n * C, C)],
                                             jnp.float32)

            accs = []
            for h0 in (0, HALF):
                ttf = [_lane_bcast(ttf16, h0 + i) for i in range(HALF)]

                @plsc.parallel_loop(0, HJ, step=8,
                                    carry=tuple(zero for _ in range(2 * HALF)))
                def p1_acc(j, acc, ttf=ttf, h0=h0, b=b):
                    out = list(acc)
                    for u in range(8):
                        sl = pl.ds((j + u) * L, L)
                        t0 = tte_v[0, sl]
                        td = tte_v[1, sl] - t0
                        for i in range(HALF):
                            t = h0 + i
                            e = (w_v[b, t, sl] + p_v[b, t, sl]
                                 + (t0 + ttf[i] * td))
                            o_v[b, t, sl] = e
                            out[2 * i] = out[2 * i] + e
                            out[2 * i + 1] = out[2 * i + 1] + e * e
                    return tuple(out)

                accs.append(p1_acc)

            @pl.when(n + 2 < NCHUNK)
            def _():
                issue_in(n + 2, b)

            for h0, acc in zip((0, HALF), accs):
                rs = []
                ms = []
                for i in range(HALF):
                    mean_v = (_lane_sum_bcast(acc[2 * i])
                              * jnp.float32(1.0 / HID))
                    msq_v = (_lane_sum_bcast(acc[2 * i + 1])
                             * jnp.float32(1.0 / HID))
                    var_v = msq_v - mean_v * mean_v + jnp.float32(1e-12)
                    r = _rsqrt_newton(var_v)
                    rs.append(r)
                    ms.append(mean_v * r)

                @plsc.parallel_loop(0, HJ, unroll=2)
                def p3_body(j, rs=rs, ms=ms, h0=h0, b=b):
                    sl = pl.ds(j * L, L)
                    g = g_v[sl]
                    bb = b_v[sl]
                    for i in range(HALF):
                        t = h0 + i
                        e = o_v[b, t, sl]
                        o_v[b, t, sl] = (e * rs[i] - ms[i]) * g + bb

            pltpu.async_copy(o_v.at[b], out_hbm.at[pl.ds(tbase, C)], sem_o[b])
        return 0

    lax.fori_loop(0, NCHUNK // 2, pair_body, 0)
    wait_out(0)
    wait_out(1)


@jax.jit
def kernel(input_ids, token_type_ids, word_embeddings, position_embeddings,
           token_type_embeddings, ln_gamma, ln_beta):
    ids = input_ids.reshape(N).astype(jnp.int32)
    tts = token_type_ids.reshape(N).astype(jnp.int32)

    emb = pl.kernel(
        _sc_body,
        out_type=jax.ShapeDtypeStruct((N, HID), jnp.float32),
        mesh=plsc.VectorSubcoreMesh(core_axis_name="c", subcore_axis_name="s"),
        scratch_types=[
            pltpu.VMEM((TPW,), jnp.int32),
            pltpu.VMEM((TPW,), jnp.int32),
            pltpu.VMEM((2, C, HID), jnp.float32),
            pltpu.VMEM((2, C, HID), jnp.float32),
            pltpu.VMEM((2, C, HID), jnp.float32),
            pltpu.VMEM((HID,), jnp.float32),
            pltpu.VMEM((HID,), jnp.float32),
            pltpu.VMEM((2, HID), jnp.float32),
            pltpu.SemaphoreType.DMA,
            pltpu.SemaphoreType.DMA,
            pltpu.SemaphoreType.DMA,
            pltpu.SemaphoreType.DMA,
            pltpu.SemaphoreType.DMA,
            pltpu.SemaphoreType.DMA,
        ],
    )(ids, tts, word_embeddings, position_embeddings, token_type_embeddings,
      ln_gamma, ln_beta)
    return emb.reshape(B, S, HID)
